# Initial kernel scaffold; baseline (speedup 1.0000x reference)
import jax
import jax.numpy as jnp
from jax.experimental import pallas as pl


def kernel(is_ligand, token_bonds, atom_to_token_map, crd_mask_L, X_L, X_gt_L):
    raise NotImplementedError("write your pallas kernel here")


# speedup vs baseline: 1614.8066x; 1614.8066x over previous
import jax
import jax.numpy as jnp
from jax.experimental import pallas as pl
from jax.experimental.pallas import tpu as pltpu

_BR = 256


def _body(lig_r, nlig_c, tb, map_r, map_c, crd_r, crd_c, xr, xc, gr, gc,
          out_w, out_l, num_acc, den_acc, tba):
    i = pl.program_id(0)
    nblocks = pl.num_programs(0)
    T = tb.shape[0]
    L = map_c.shape[1]

    @pl.when(i == 0)
    def _init():
        num_acc[0, 0] = 0.0
        den_acc[0, 0] = 0.0
        plf = tb[...] * lig_r[...] * nlig_c[...]
        tok_iota = jax.lax.broadcasted_iota(jnp.int32, (T, L), 0)
        onehot_c = (tok_iota == map_c[...]).astype(jnp.bfloat16)
        tba[...] = jax.lax.dot_general(
            plf.astype(jnp.bfloat16), onehot_c,
            (((1,), (0,)), ((), ())),
            preferred_element_type=jnp.float32).astype(jnp.bfloat16)

    lane_iota = jax.lax.broadcasted_iota(jnp.int32, (_BR, T), 1)
    onehot_r = (lane_iota == map_r[...]).astype(jnp.bfloat16)
    bonds = jax.lax.dot_general(
        onehot_r, tba[...], (((1,), (0,)), ((), ())),
        preferred_element_type=jnp.float32)

    xr_ = xr[...]
    xc_ = xc[...]
    gr_ = gr[...]
    gc_ = gc[...]
    rn = jnp.sum(xr_ * xr_, axis=1, keepdims=True)
    cn = jnp.sum(xc_ * xc_, axis=0, keepdims=True)
    pd2 = rn + cn - 2.0 * jax.lax.dot_general(
        xr_, xc_, (((1,), (0,)), ((), ())), preferred_element_type=jnp.float32)
    grn = jnp.sum(gr_ * gr_, axis=1, keepdims=True)
    gcn = jnp.sum(gc_ * gc_, axis=0, keepdims=True)
    gd2 = grn + gcn - 2.0 * jax.lax.dot_general(
        gr_, gc_, (((1,), (0,)), ((), ())), preferred_element_type=jnp.float32)
    pd2 = jnp.maximum(pd2, 0.0)
    gd2 = jnp.maximum(gd2, 0.0)
    pd = jnp.sqrt(pd2)
    gd = jnp.sqrt(gd2)

    maskf = (bonds * (crd_r[...] * crd_c[...])) * (gd < 2.4).astype(jnp.float32)
    diff = pd - gd
    num_acc[0, 0] += jnp.sum(diff * diff * maskf)
    den_acc[0, 0] += jnp.sum(maskf)

    @pl.when(i == nblocks - 1)
    def _fin():
        loss = num_acc[0, 0] / jnp.maximum(den_acc[0, 0], 1.0)
        out_w[...] = jnp.full((1, 1), loss, dtype=jnp.float32)
        out_l[...] = jnp.full((1, 1), loss, dtype=jnp.float32)


def kernel(is_ligand, token_bonds, atom_to_token_map, crd_mask_L, X_L, X_gt_L):
    T = is_ligand.shape[0]
    L = atom_to_token_map.shape[0]
    ligf = is_ligand.astype(jnp.float32)
    crdf = crd_mask_L[0].astype(jnp.float32)
    mapi = atom_to_token_map.astype(jnp.int32)
    x = X_L[0]
    g = X_gt_L[0]

    full = lambda shape: pl.BlockSpec(shape, lambda i: (0,) * len(shape))
    grid = L // _BR
    out_w, out_l = pl.pallas_call(
        _body,
        grid=(grid,),
        in_specs=[
            full((T, 1)),
            full((1, T)),
            full((T, T)),
            pl.BlockSpec((_BR, 1), lambda i: (i, 0)),
            full((1, L)),
            pl.BlockSpec((_BR, 1), lambda i: (i, 0)),
            full((1, L)),
            pl.BlockSpec((_BR, 3), lambda i: (i, 0)),
            full((3, L)),
            pl.BlockSpec((_BR, 3), lambda i: (i, 0)),
            full((3, L)),
        ],
        out_specs=[full((1, 1)), full((1, 1))],
        out_shape=[jax.ShapeDtypeStruct((1, 1), jnp.float32),
                   jax.ShapeDtypeStruct((1, 1), jnp.float32)],
        scratch_shapes=[
            pltpu.SMEM((1, 1), jnp.float32),
            pltpu.SMEM((1, 1), jnp.float32),
            pltpu.VMEM((T, L), jnp.bfloat16),
        ],
    )(
        ligf.reshape(T, 1),
        (1.0 - ligf).reshape(1, T),
        token_bonds.astype(jnp.float32),
        mapi.reshape(L, 1),
        mapi.reshape(1, L),
        crdf.reshape(L, 1),
        crdf.reshape(1, L),
        x,
        x.T,
        g,
        g.T,
    )
    loss = out_l.reshape(())
    return (1.0 * out_w.reshape(()), loss)
